# Initial kernel scaffold; baseline (speedup 1.0000x reference)
import jax
import jax.numpy as jnp
from jax.experimental import pallas as pl


def kernel(x, edge_index, virus_ids, W1, b1, W2, b2, fc1_w, fc1_b, fc2_w, fc2_b):
    raise NotImplementedError("write your pallas kernel here")


# speedup vs baseline: 2.6709x; 2.6709x over previous
import functools

import jax
import jax.numpy as jnp
from jax.experimental import pallas as pl
from jax.experimental.pallas import tpu as pltpu

N = 10000
NPAD = 10240
E = 320000
DIN = 128
H1 = 512
H2 = 256
V = 500
VPAD = 512
BN = 1280


def _tc1_body(degp_ref, x_ref, dinv_ref, y_ref):
    deg = degp_ref[0] + degp_ref[1] + 1.0
    dinv = jax.lax.rsqrt(deg)
    dinv_ref[0:N, :] = dinv
    dinv_ref[N:NPAD, :] = jnp.zeros((NPAD - N, 1), jnp.float32)
    y_ref[0:N, :] = dinv * x_ref[...]
    y_ref[N:NPAD, :] = jnp.zeros((NPAD - N, DIN), jnp.float32)


def _tc1(deg_parts, x):
    return pl.pallas_call(
        _tc1_body,
        out_shape=[
            jax.ShapeDtypeStruct((NPAD, 1), jnp.float32),
            jax.ShapeDtypeStruct((NPAD, DIN), jnp.float32),
        ],
    )(deg_parts, x)


def _tc2_body(pp_ref, y_ref, dinv_ref, w1_ref, b1_ref, w2_ref, cpt_ref,
              cnt_ref, b2_ref, f1w_ref, f1b_ref, f2w_ref, f2b_ref,
              out_ref, acc_ref):
    i = pl.program_id(0)
    nsteps = pl.num_programs(0)
    dinv = dinv_ref[...]
    p1 = dinv * (pp_ref[0] + pp_ref[1] + y_ref[...])
    h = jnp.maximum(jnp.dot(p1, w1_ref[...],
                            preferred_element_type=jnp.float32)
                    + b1_ref[...], 0.0)
    mp = dinv * jnp.dot(h, w2_ref[...],
                        preferred_element_type=jnp.float32)
    contrib = jax.lax.dot_general(
        cpt_ref[0] + cpt_ref[1], mp,
        (((1,), (0,)), ((), ())),
        preferred_element_type=jnp.float32)

    @pl.when(i == 0)
    def _init():
        acc_ref[...] = contrib

    @pl.when(i > 0)
    def _accum():
        acc_ref[...] += contrib

    @pl.when(i == nsteps - 1)
    def _final():
        cnt = cnt_ref[...]
        agg = acc_ref[...] / jnp.maximum(cnt, 1.0)
        agg = agg + b2_ref[...] * (cnt > 0.0).astype(jnp.float32)
        z = jnp.maximum(jnp.dot(agg, f1w_ref[...],
                                preferred_element_type=jnp.float32)
                        + f1b_ref[...], 0.0)
        o = jnp.dot(z, f2w_ref[...],
                    preferred_element_type=jnp.float32) + f2b_ref[...]
        out_ref[...] = o[0:V, :]


def _tc2(pp, y, dinv, W1, b1, W2, cpt, counts, b2, f1w, f1b, f2w, f2b):
    nsteps = NPAD // BN
    grid = (nsteps,)
    return pl.pallas_call(
        _tc2_body,
        grid=grid,
        in_specs=[
            pl.BlockSpec((2, BN, DIN), lambda i: (0, i, 0)),
            pl.BlockSpec((BN, DIN), lambda i: (i, 0)),
            pl.BlockSpec((BN, 1), lambda i: (i, 0)),
            pl.BlockSpec((DIN, H1), lambda i: (0, 0)),
            pl.BlockSpec((1, H1), lambda i: (0, 0)),
            pl.BlockSpec((H1, H2), lambda i: (0, 0)),
            pl.BlockSpec((2, VPAD, BN), lambda i: (0, 0, i)),
            pl.BlockSpec((VPAD, 1), lambda i: (0, 0)),
            pl.BlockSpec((1, H2), lambda i: (0, 0)),
            pl.BlockSpec((H2, H2), lambda i: (0, 0)),
            pl.BlockSpec((1, H2), lambda i: (0, 0)),
            pl.BlockSpec((H2, 1), lambda i: (0, 0)),
            pl.BlockSpec((1, 1), lambda i: (0, 0)),
        ],
        out_specs=pl.BlockSpec((V, 1), lambda i: (0, 0)),
        out_shape=jax.ShapeDtypeStruct((V, 1), jnp.float32),
        scratch_shapes=[pltpu.VMEM((VPAD, H2), jnp.float32)],
    )(pp, y, dinv, W1, b1, W2, cpt, counts, b2, f1w, f1b, f2w, f2b)


def kernel(x, edge_index, virus_ids, W1, b1, W2, b2, fc1_w, fc1_b, fc2_w, fc2_b):
    src = edge_index[0]
    dst = edge_index[1]

    half = E // 2
    dega = jnp.zeros((N,), jnp.float32).at[dst[:half]].add(1.0)
    degb = jnp.zeros((N,), jnp.float32).at[dst[half:]].add(1.0)
    deg_parts = jnp.stack([dega, degb])[:, :, None]
    counts = jnp.zeros((VPAD,), jnp.float32).at[virus_ids].add(1.0)
    counts = counts[:, None]

    dinv_p, y = _tc1(deg_parts, x)
    dinv_flat = dinv_p[:, 0]

    ppa = jnp.zeros((NPAD, DIN), jnp.float32).at[dst[:half]].add(y[src[:half]])
    ppb = jnp.zeros((NPAD, DIN), jnp.float32).at[dst[half:]].add(y[src[half:]])
    pp = jnp.stack([ppa, ppb])

    vdst = virus_ids[dst]
    cpa = (jnp.zeros((VPAD, NPAD), jnp.float32)
           .at[vdst[:half], src[:half]].add(dinv_flat[dst[:half]]))
    cpa = cpa.at[virus_ids, jnp.arange(N)].add(dinv_flat[:N])
    cpb = (jnp.zeros((VPAD, NPAD), jnp.float32)
           .at[vdst[half:], src[half:]].add(dinv_flat[dst[half:]]))
    cpt = jnp.stack([cpa, cpb])

    return _tc2(pp, y, dinv_p, W1, b1[None, :], W2, cpt, counts,
                b2[None, :], fc1_w, fc1_b[None, :], fc2_w, fc2_b[None, :])


# speedup vs baseline: 3.4984x; 1.3098x over previous
import functools

import jax
import jax.numpy as jnp
from jax import lax
from jax.experimental import pallas as pl
from jax.experimental.pallas import tpu as pltpu
from jax.experimental.pallas import tpu_sc as plsc

N = 10000
NPAD = 10240
E = 320000
DIN = 128
H1 = 512
H2 = 256
V = 500
VPAD = 512
BN = 1280

NC = 2
NS = 16
NT = NC * NS
EPT = E // NT
EPTP = 10240
ECH = EPTP // 128
NPT = NPAD // NT
EPTC = 11264
ECHC = EPTC // 128
CGRP = 8
CCH = 16
CROWS = VPAD // CCH
CWORDS = CROWS * NPAD
CPW = CWORDS // NS
DUMW = CWORDS
DSH = NPAD // NS
CSH = VPAD // NS


def _sc_mesh():
    return plsc.VectorSubcoreMesh(core_axis_name="c", subcore_axis_name="s")


def _sca_body(dstfh, vidsfh, deg_out, cnt_out, dloc, cloc, dstv, vidv,
              rrow, obuf, shd, shc):
    c = lax.axis_index("c")
    s = lax.axis_index("s")
    w = c * NS + s
    zero16 = jnp.zeros((16,), jnp.float32)
    one16 = jnp.full((16,), 1.0, jnp.float32)

    def zd(i, _):
        dloc[pl.ds(i * 16, 16)] = zero16
        return 0
    lax.fori_loop(0, NPAD // 16, zd, 0)
    for i in range(VPAD // 16):
        cloc[pl.ds(i * 16, 16)] = zero16

    pltpu.sync_copy(dstfh.at[w], dstv)
    pltpu.sync_copy(vidsfh.at[w], vidv)

    def ed(i, _):
        idx = dstv[pl.ds(i * 16, 16)]
        plsc.addupdate_scatter(dloc, [idx], one16)
        return 0
    lax.fori_loop(0, EPTP // 16, ed, 0)

    def vd(i, _):
        idx = vidv[pl.ds(i * 16, 16)]
        plsc.addupdate_scatter(cloc, [idx], one16)
        return 0
    lax.fori_loop(0, 384 // 16, vd, 0)

    pltpu.sync_copy(dloc, shd.at[s])
    pltpu.sync_copy(cloc, shc.at[s])
    plsc.subcore_barrier()

    pltpu.sync_copy(shd.at[0, pl.ds(s * DSH, DSH)], obuf)
    for j in range(1, NS):
        pltpu.sync_copy(shd.at[j, pl.ds(s * DSH, DSH)], rrow)

        def addr(k, _):
            obuf[pl.ds(k * 16, 16)] += rrow[pl.ds(k * 16, 16)]
            return 0
        lax.fori_loop(0, DSH // 16, addr, 0)
    pltpu.sync_copy(obuf, deg_out.at[c, pl.ds(s * DSH, DSH)])

    pltpu.sync_copy(shc.at[0, pl.ds(s * CSH, CSH)], obuf.at[pl.ds(0, CSH)])
    for j in range(1, NS):
        pltpu.sync_copy(shc.at[j, pl.ds(s * CSH, CSH)], rrow.at[pl.ds(0, CSH)])
        for k in range(CSH // 16):
            obuf[pl.ds(k * 16, 16)] += rrow[pl.ds(k * 16, 16)]
    pltpu.sync_copy(obuf.at[pl.ds(0, CSH)], cnt_out.at[c, pl.ds(s * CSH, CSH)])


def _sca(dstfh, vidsfh):
    f = pl.kernel(
        _sca_body,
        out_type=[
            jax.ShapeDtypeStruct((NC, NPAD), jnp.float32),
            jax.ShapeDtypeStruct((NC, VPAD), jnp.float32),
        ],
        mesh=_sc_mesh(),
        scratch_types=[
            pltpu.VMEM((NPAD,), jnp.float32),
            pltpu.VMEM((VPAD,), jnp.float32),
            pltpu.VMEM((EPTP,), jnp.int32),
            pltpu.VMEM((384,), jnp.int32),
            pltpu.VMEM((DSH,), jnp.float32),
            pltpu.VMEM((DSH,), jnp.float32),
            pltpu.VMEM_SHARED((NS, NPAD), jnp.float32),
            pltpu.VMEM_SHARED((NS, VPAD), jnp.float32),
        ],
        compiler_params=pltpu.CompilerParams(needs_layout_passes=False),
    )
    return f(dstfh, vidsfh)


PH = NPAD // 2
PSH = PH // NS


def _scb_body(y_hbm, srcf, dstf, pp_out,
              srcv, dstv, rowsa, rowsb, zbuf, ig, p1sh, sema, semb):
    c = lax.axis_index("c")
    s = lax.axis_index("s")
    w = c * NS + s
    zero16 = jnp.zeros((16,), jnp.float32)

    pltpu.sync_copy(srcf.at[w], srcv)
    pltpu.sync_copy(dstf.at[w], dstv)

    for r in range(8):
        for k in range(8):
            zbuf[r, pl.ds(k * 16, 16)] = zero16

    for h in range(2):
        base = h * PH

        def zb(i, _):
            pltpu.sync_copy(zbuf, p1sh.at[pl.ds(s * PSH + i * 8, 8)])
            return 0
        lax.fori_loop(0, PSH // 8, zb, 0)
        plsc.subcore_barrier()

        pltpu.async_copy(y_hbm.at[srcv.at[pl.ds(0, 128)]], rowsa, sema)

        def chunk(i, _):
            even = jnp.bitwise_and(i, 1) == 0

            def stage_scatter(rows):
                for k in range(8):
                    d16 = dstv[pl.ds(i * 128 + k * 16, 16)] - base
                    m = jnp.logical_and(d16 >= 0, d16 < PH)
                    ig[0, pl.ds(k * 16, 16)] = jnp.where(m, d16, PH)
                pltpu.sync_copy(rows, p1sh.at[ig.at[0]], add=True)

            @pl.when(even)
            def _():
                pltpu.make_async_copy(
                    y_hbm.at[srcv.at[pl.ds(i * 128, 128)]], rowsa, sema).wait()

                @pl.when(i + 1 < ECH)
                def _():
                    pltpu.async_copy(
                        y_hbm.at[srcv.at[pl.ds((i + 1) * 128, 128)]],
                        rowsb, semb)
                stage_scatter(rowsa)

            @pl.when(jnp.logical_not(even))
            def _():
                pltpu.make_async_copy(
                    y_hbm.at[srcv.at[pl.ds(i * 128, 128)]], rowsb, semb).wait()

                @pl.when(i + 1 < ECH)
                def _():
                    pltpu.async_copy(
                        y_hbm.at[srcv.at[pl.ds((i + 1) * 128, 128)]],
                        rowsa, sema)
                stage_scatter(rowsb)
            return 0
        lax.fori_loop(0, ECH, chunk, 0)
        plsc.subcore_barrier()
        pltpu.sync_copy(p1sh.at[pl.ds(s * PSH, PSH)],
                        pp_out.at[c, pl.ds(base + s * PSH, PSH)])
        plsc.subcore_barrier()


def _scb(y, srcf, dstf):
    f = pl.kernel(
        _scb_body,
        out_type=jax.ShapeDtypeStruct((NC, NPAD, DIN), jnp.float32),
        mesh=_sc_mesh(),
        scratch_types=[
            pltpu.VMEM((EPTP,), jnp.int32),
            pltpu.VMEM((EPTP,), jnp.int32),
            pltpu.VMEM((128, DIN), jnp.float32),
            pltpu.VMEM((128, DIN), jnp.float32),
            pltpu.VMEM((8, DIN), jnp.float32),
            pltpu.VMEM((8, 128), jnp.int32),
            pltpu.VMEM_SHARED((PH + 8, DIN), jnp.float32),
            pltpu.SemaphoreType.DMA,
            pltpu.SemaphoreType.DMA,
        ],
        compiler_params=pltpu.CompilerParams(needs_layout_passes=False),
    )
    return f(y, srcf, dstf)


def _scc_body(dinv_hbm, vids_hbm, srcf, dstf, cpt_out,
              dinl, vidl, srcv, dstv, valb, gvb, idxg, valg, zbuf, cpsh, sem):
    c = lax.axis_index("c")
    s = lax.axis_index("s")
    w = c * NS + s
    zero16 = jnp.zeros((16,), jnp.float32)

    pltpu.sync_copy(dinv_hbm, dinl)
    pltpu.sync_copy(vids_hbm, vidl)
    pltpu.sync_copy(srcf.at[w], srcv)
    pltpu.sync_copy(dstf.at[w], dstv)

    def pre(i, _):
        d = dstv[pl.ds(i * 16, 16)]
        sq = srcv[pl.ds(i * 16, 16)]
        vd = plsc.load_gather(vidl, [d])
        val = plsc.load_gather(dinl, [d])
        gvb[pl.ds(i * 16, 16)] = vd * NPAD + sq
        valb[pl.ds(i * 16, 16)] = val
        return 0
    lax.fori_loop(0, EPTC // 16, pre, 0)

    def zz(i, _):
        zbuf[pl.ds(i * 16, 16)] = zero16
        return 0
    lax.fori_loop(0, 4096 // 16, zz, 0)

    for c4 in range(CCH):
        for j in range(CPW // 4096):
            pltpu.sync_copy(zbuf, cpsh.at[pl.ds(s * CPW + j * 4096, 4096)])
        plsc.subcore_barrier()

        def grp(g, _):
            for u in range(CGRP):
                for k in range(8):
                    off = g * CGRP * 128 + u * 128 + k * 16
                    loc = gvb[pl.ds(off, 16)] - c4 * CWORDS
                    m = jnp.logical_and(loc >= 0, loc < CWORDS)
                    idxg[u, pl.ds(k * 16, 16)] = jnp.where(m, loc, DUMW)
                    valg[u, pl.ds(k * 16, 16)] = valb[pl.ds(off, 16)]
            ds_ = []
            for u in range(CGRP):
                ds_.append(pltpu.async_copy(
                    valg.at[u], cpsh.at[idxg.at[u]], sem, add=True))
            for dsc in ds_:
                dsc.wait()
            return 0
        lax.fori_loop(0, ECHC // CGRP, grp, 0)
        plsc.subcore_barrier()
        pltpu.sync_copy(
            cpsh.at[pl.ds(s * CPW, CPW)],
            cpt_out.at[c, pl.ds(c4 * CWORDS + s * CPW, CPW)])


def _scc(dinv_flat, vids_full, srcf, dstf):
    f = pl.kernel(
        _scc_body,
        out_type=jax.ShapeDtypeStruct((NC, VPAD * NPAD), jnp.float32),
        mesh=_sc_mesh(),
        scratch_types=[
            pltpu.VMEM((NPAD,), jnp.float32),
            pltpu.VMEM((NPAD,), jnp.int32),
            pltpu.VMEM((EPTC,), jnp.int32),
            pltpu.VMEM((EPTC,), jnp.int32),
            pltpu.VMEM((EPTC,), jnp.float32),
            pltpu.VMEM((EPTC,), jnp.int32),
            pltpu.VMEM((CGRP, 128), jnp.int32),
            pltpu.VMEM((CGRP, 128), jnp.float32),
            pltpu.VMEM((4096,), jnp.float32),
            pltpu.VMEM_SHARED((CWORDS + 64,), jnp.float32),
            pltpu.SemaphoreType.DMA,
        ],
        compiler_params=pltpu.CompilerParams(needs_layout_passes=False),
    )
    return f(dinv_flat, vids_full, srcf, dstf)


def _tc1_body(degp_ref, cntp_ref, x_ref, dinv_ref, y_ref, cnt_ref):
    deg = degp_ref[0, 0:N] + degp_ref[1, 0:N] + 1.0
    r = jax.lax.rsqrt(deg)
    r = r * (1.5 - 0.5 * deg * r * r)
    dinv = r * (1.5 - 0.5 * deg * r * r)
    dinv_ref[0:N, :] = dinv
    dinv_ref[N:NPAD, :] = jnp.zeros((NPAD - N, 1), jnp.float32)
    y_ref[0:N, :] = dinv * x_ref[...]
    y_ref[N:NPAD, :] = jnp.zeros((NPAD - N, DIN), jnp.float32)
    cnt_ref[...] = cntp_ref[0] + cntp_ref[1]


def _tc1(deg_parts, cnt_parts, x):
    return pl.pallas_call(
        _tc1_body,
        out_shape=[
            jax.ShapeDtypeStruct((NPAD, 1), jnp.float32),
            jax.ShapeDtypeStruct((NPAD, DIN), jnp.float32),
            jax.ShapeDtypeStruct((VPAD, 1), jnp.float32),
        ],
    )(deg_parts, cnt_parts, x)


def _tc2_body(pp_ref, y_ref, dinv_ref, w1_ref, b1_ref, w2_ref, cpt_ref,
              cnt_ref, b2_ref, f1w_ref, f1b_ref, f2w_ref, f2b_ref,
              out_ref, acc_ref):
    i = pl.program_id(0)
    nsteps = pl.num_programs(0)
    dinv = dinv_ref[...]
    p1 = dinv * (pp_ref[0] + pp_ref[1] + y_ref[...])
    h = jnp.maximum(jnp.dot(p1, w1_ref[...],
                            preferred_element_type=jnp.float32,
                            precision=jax.lax.Precision.HIGHEST)
                    + b1_ref[...], 0.0)
    mp = dinv * jnp.dot(h, w2_ref[...],
                        preferred_element_type=jnp.float32,
                        precision=jax.lax.Precision.HIGHEST)
    contrib = jax.lax.dot_general(
        cpt_ref[0] + cpt_ref[1], mp,
        (((1,), (0,)), ((), ())),
        preferred_element_type=jnp.float32,
        precision=jax.lax.Precision.HIGHEST)

    @pl.when(i == 0)
    def _init():
        acc_ref[...] = contrib

    @pl.when(i > 0)
    def _accum():
        acc_ref[...] += contrib

    @pl.when(i == nsteps - 1)
    def _final():
        cnt = cnt_ref[...]
        agg = acc_ref[...] / jnp.maximum(cnt, 1.0)
        agg = agg + b2_ref[...] * (cnt > 0.0).astype(jnp.float32)
        z = jnp.maximum(jnp.dot(agg, f1w_ref[...],
                                preferred_element_type=jnp.float32,
                                precision=jax.lax.Precision.HIGHEST)
                        + f1b_ref[...], 0.0)
        o = jnp.dot(z, f2w_ref[...],
                    preferred_element_type=jnp.float32,
                    precision=jax.lax.Precision.HIGHEST) + f2b_ref[...]
        out_ref[...] = o[0:V, :]


def _tc2(pp, y, dinv, W1, b1, W2, cpt, counts, b2, f1w, f1b, f2w, f2b):
    nsteps = NPAD // BN
    grid = (nsteps,)
    return pl.pallas_call(
        _tc2_body,
        grid=grid,
        in_specs=[
            pl.BlockSpec((2, BN, DIN), lambda i: (0, i, 0)),
            pl.BlockSpec((BN, DIN), lambda i: (i, 0)),
            pl.BlockSpec((BN, 1), lambda i: (i, 0)),
            pl.BlockSpec((DIN, H1), lambda i: (0, 0)),
            pl.BlockSpec((1, H1), lambda i: (0, 0)),
            pl.BlockSpec((H1, H2), lambda i: (0, 0)),
            pl.BlockSpec((2, VPAD, BN), lambda i: (0, 0, i)),
            pl.BlockSpec((VPAD, 1), lambda i: (0, 0)),
            pl.BlockSpec((1, H2), lambda i: (0, 0)),
            pl.BlockSpec((H2, H2), lambda i: (0, 0)),
            pl.BlockSpec((1, H2), lambda i: (0, 0)),
            pl.BlockSpec((H2, 1), lambda i: (0, 0)),
            pl.BlockSpec((1, 1), lambda i: (0, 0)),
        ],
        out_specs=pl.BlockSpec((V, 1), lambda i: (0, 0)),
        out_shape=jax.ShapeDtypeStruct((V, 1), jnp.float32),
        scratch_shapes=[pltpu.VMEM((VPAD, H2), jnp.float32)],
    )(pp, y, dinv, W1, b1, W2, cpt, counts, b2, f1w, f1b, f2w, f2b)


def kernel(x, edge_index, virus_ids, W1, b1, W2, b2, fc1_w, fc1_b, fc2_w, fc2_b):
    src = edge_index[0]
    dst = edge_index[1]

    srcf = jnp.pad(src.reshape(NT, EPT), ((0, 0), (0, EPTP - EPT)),
                   constant_values=N)
    dstf = jnp.pad(dst.reshape(NT, EPT), ((0, 0), (0, EPTP - EPT)),
                   constant_values=N)
    vidsf = jnp.pad(virus_ids, (0, NT * 384 - N),
                    constant_values=VPAD).reshape(NT, 384)

    deg_parts, cnt_parts = _sca(dstf, vidsf)
    dinv_p, y, counts = _tc1(deg_parts[:, :, None],
                             cnt_parts[:, :, None], x)
    dinv_flat = dinv_p[:, 0]
    pp = _scb(y, srcf, dstf)

    loop_ids = jnp.arange(N, dtype=src.dtype)
    srcc = jnp.pad(jnp.concatenate([src, loop_ids]), (0, NT * EPTC - E - N),
                   constant_values=NPAD - 1).reshape(NT, EPTC)
    dstc = jnp.pad(jnp.concatenate([dst, loop_ids]), (0, NT * EPTC - E - N),
                   constant_values=NPAD - 1).reshape(NT, EPTC)
    vids_full = jnp.pad(virus_ids, (0, NPAD - N), constant_values=V)
    cpt = _scc(dinv_flat, vids_full, srcc, dstc).reshape(NC, VPAD, NPAD)

    return _tc2(pp, y, dinv_p, W1, b1[None, :], W2, cpt, counts,
                b2[None, :], fc1_w, fc1_b[None, :], fc2_w, fc2_b[None, :])
